# baseline (speedup 1.0000x reference)
import functools

import jax
import jax.numpy as jnp
from jax import lax
from jax.experimental import pallas as pl
from jax.experimental.pallas import tpu as pltpu
from jax.experimental.pallas import tpu_sc as plsc

_B = 4096
_F = 26
_V = 100000
_D = 32

_NC = 2
_NS = 16
_NW = _NC * _NS


_CHUNKS = ((0, 24960), (24960, 24960), (49920, 24960), (74880, 25120))


def _sc_gather_body(tabt_hbm, idxt_hbm, outt_hbm,
                    buf0_v, buf1_v, buf2_v, buf3_v, idx_v, row_v,
                    sem0, sem1, sem2, sem3):
    d = lax.axis_index("c") * _NS + lax.axis_index("s")
    bufs = (buf0_v, buf1_v, buf2_v, buf3_v)
    sems = (sem0, sem1, sem2, sem3)

    def start(k, f):
        off, sz = _CHUNKS[k]
        return pltpu.async_copy(tabt_hbm.at[f, d, pl.ds(off, sz)], bufs[k], sems[k])

    for k in range(4):
        start(k, 0)
    pltpu.sync_copy(idxt_hbm.at[0], idx_v)

    def step(f, carry):
        for k in range(4):
            off, sz = _CHUNKS[k]
            pltpu.make_async_copy(
                tabt_hbm.at[f, d, pl.ds(off, sz)], bufs[k], sems[k]
            ).wait()

            def gk(i, c2, k=k, off=off, sz=sz):
                sl = pl.ds(i * 16, 16)
                v = idx_v[sl]
                m = (v >= off) & (v < off + sz)
                gv = plsc.load_gather(bufs[k], [jnp.where(m, v - off, 0)])
                gv = jnp.where(m, gv, 0.0)
                if k == 0:
                    row_v[sl] = gv
                else:
                    row_v[sl] = row_v[sl] + gv
                return c2

            lax.fori_loop(0, _B // 16, gk, 0)

            @pl.when(f + 1 < _F)
            def _(k=k):
                start(k, f + 1)

        pltpu.sync_copy(row_v, outt_hbm.at[f, d])

        @pl.when(f + 1 < _F)
        def _():
            pltpu.sync_copy(idxt_hbm.at[f + 1], idx_v)

        return carry

    lax.fori_loop(0, _F, step, 0)


@functools.cache
def _sc_gather():
    return pl.kernel(
        _sc_gather_body,
        out_type=jax.ShapeDtypeStruct((_F, _D, _B), jnp.float32),
        mesh=plsc.VectorSubcoreMesh(
            core_axis_name="c", subcore_axis_name="s", num_cores=_NC, num_subcores=_NS
        ),
        scratch_types=[
            pltpu.VMEM((_CHUNKS[0][1],), jnp.float32),
            pltpu.VMEM((_CHUNKS[1][1],), jnp.float32),
            pltpu.VMEM((_CHUNKS[2][1],), jnp.float32),
            pltpu.VMEM((_CHUNKS[3][1],), jnp.float32),
            pltpu.VMEM((_B,), jnp.int32),
            pltpu.VMEM((_B,), jnp.float32),
            pltpu.SemaphoreType.DMA,
            pltpu.SemaphoreType.DMA,
            pltpu.SemaphoreType.DMA,
            pltpu.SemaphoreType.DMA,
        ],
        compiler_params=pltpu.CompilerParams(
            use_tc_tiling_on_sc=True, needs_layout_passes=False
        ),
    )


_BB = 512


def _tc_body(dense_ref, spt_ref,
             dW1, db1, dW2, db2, dW3, db3,
             oW1, ob1, oW2, ob2, oW3, ob3,
             out_ref):
    relu = lambda v: jnp.maximum(v, 0.0)
    h = relu(jnp.dot(dense_ref[...], dW1[...], preferred_element_type=jnp.float32) + db1[...])
    h = relu(jnp.dot(h, dW2[...], preferred_element_type=jnp.float32) + db2[...])
    de = relu(jnp.dot(h, dW3[...], preferred_element_type=jnp.float32) + db3[...])

    sp = jnp.transpose(spt_ref[...])
    c3 = jnp.concatenate([de, sp], axis=1).reshape(_BB, _F + 1, _D)
    g = lax.dot_general(
        c3, c3,
        dimension_numbers=(((2,), (2,)), ((0,), (0,))),
        preferred_element_type=jnp.float32,
    )
    gf = g.reshape(_BB, (_F + 1) * (_F + 1))
    h = relu(
        jnp.dot(de, oW1[: _D], preferred_element_type=jnp.float32)
        + jnp.dot(gf, oW1[_D:], preferred_element_type=jnp.float32)
        + ob1[...]
    )
    h = relu(jnp.dot(h, oW2[...], preferred_element_type=jnp.float32) + ob2[...])
    out_ref[...] = jnp.dot(h, oW3[...], preferred_element_type=jnp.float32) + ob3[...]


def _tc_forward(dense, spt, dW1, db1, dW2, db2, dW3, db3, oW1, ob1, oW2, ob2, oW3, ob3,
                interpret=False):
    full = lambda shape: pl.BlockSpec(shape, lambda i: (0,) * len(shape))
    grid = _B // _BB
    return pl.pallas_call(
        _tc_body,
        grid=(grid,),
        in_specs=[
            pl.BlockSpec((_BB, 13), lambda i: (i, 0)),
            pl.BlockSpec((_F * _D, _BB), lambda i: (0, i)),
            full(dW1.shape), full(db1.shape), full(dW2.shape), full(db2.shape),
            full(dW3.shape), full(db3.shape),
            full(oW1.shape), full(ob1.shape), full(oW2.shape), full(ob2.shape),
            full(oW3.shape), full(ob3.shape),
        ],
        out_specs=pl.BlockSpec((_BB, 1), lambda i: (i, 0)),
        out_shape=jax.ShapeDtypeStruct((_B, 1), jnp.float32),
        interpret=interpret,
    )(dense, spt, dW1, db1, dW2, db2, dW3, db3, oW1, ob1, oW2, ob2, oW3, ob3)


def _permute_over_w1(oW1):
    n_idx, m_idx = jnp.triu_indices(_F + 1, k=1)
    w_int = 0.5 * oW1[_D:]
    wp = jnp.zeros(((_F + 1) * (_F + 1), oW1.shape[1]), oW1.dtype)
    wp = wp.at[n_idx * (_F + 1) + m_idx].set(w_int)
    wp = wp.at[m_idx * (_F + 1) + n_idx].set(w_int)
    return jnp.concatenate([oW1[:_D], wp], axis=0)


def kernel(dense_features, sparse_indices, tables,
           dW1, db1, dW2, db2, dW3, db3,
           oW1, ob1, oW2, ob2, oW3, ob3):
    tab_t = jnp.transpose(tables, (0, 2, 1))
    idx_t = jnp.transpose(sparse_indices)
    out_t = _sc_gather()(tab_t, idx_t)
    spt = out_t.reshape(_F * _D, _B)
    return _tc_forward(
        dense_features, spt,
        dW1, db1.reshape(1, -1), dW2, db2.reshape(1, -1), dW3, db3.reshape(1, -1),
        _permute_over_w1(oW1), ob1.reshape(1, -1),
        oW2, ob2.reshape(1, -1), oW3, ob3.reshape(1, -1),
    )


# speedup vs baseline: 1.3109x; 1.3109x over previous
import functools

import jax
import jax.numpy as jnp
from jax import lax
from jax.experimental import pallas as pl
from jax.experimental.pallas import tpu as pltpu
from jax.experimental.pallas import tpu_sc as plsc

_B = 4096
_F = 26
_V = 100000
_D = 32

_NC = 2
_NS = 16
_NW = _NC * _NS


_VA = 49920
_VB = _V - _VA


def _sc_gather_body(tabt_hbm, idxt_hbm, outt_hbm, bufa_v, bufb_v, idx_v, row_v,
                    sema, semb):
    d = lax.axis_index("c") * _NS + lax.axis_index("s")

    def start_a(f):
        return pltpu.async_copy(tabt_hbm.at[f, d, pl.ds(0, _VA)], bufa_v, sema)

    def start_b(f):
        return pltpu.async_copy(tabt_hbm.at[f, d, pl.ds(_VA, _VB)], bufb_v, semb)

    start_a(0)
    start_b(0)
    pltpu.sync_copy(idxt_hbm.at[0], idx_v)

    def step(f, carry):
        pltpu.make_async_copy(tabt_hbm.at[f, d, pl.ds(0, _VA)], bufa_v, sema).wait()

        def g1(i, c2):
            sl = pl.ds(i * 16, 16)
            v = idx_v[sl]
            m = v < _VA
            ga = plsc.load_gather(bufa_v, [jnp.where(m, v, 0)])
            row_v[sl] = jnp.where(m, ga, 0.0)
            return c2

        lax.fori_loop(0, _B // 16, g1, 0)

        pltpu.make_async_copy(tabt_hbm.at[f, d, pl.ds(_VA, _VB)], bufb_v, semb).wait()

        @pl.when(f + 1 < _F)
        def _():
            start_a(f + 1)

        def g2(i, c2):
            sl = pl.ds(i * 16, 16)
            v = idx_v[sl]
            m = v >= _VA
            gb = plsc.load_gather(bufb_v, [jnp.where(m, v - _VA, 0)])
            row_v[sl] = row_v[sl] + jnp.where(m, gb, 0.0)
            return c2

        lax.fori_loop(0, _B // 16, g2, 0)
        pltpu.sync_copy(row_v, outt_hbm.at[f, d])

        @pl.when(f + 1 < _F)
        def _():
            start_b(f + 1)
            pltpu.sync_copy(idxt_hbm.at[f + 1], idx_v)

        return carry

    lax.fori_loop(0, _F, step, 0)


@functools.cache
def _sc_gather():
    return pl.kernel(
        _sc_gather_body,
        out_type=jax.ShapeDtypeStruct((_F, _D, _B), jnp.float32),
        mesh=plsc.VectorSubcoreMesh(
            core_axis_name="c", subcore_axis_name="s", num_cores=_NC, num_subcores=_NS
        ),
        scratch_types=[
            pltpu.VMEM((_VA,), jnp.float32),
            pltpu.VMEM((_VB,), jnp.float32),
            pltpu.VMEM((_B,), jnp.int32),
            pltpu.VMEM((_B,), jnp.float32),
            pltpu.SemaphoreType.DMA,
            pltpu.SemaphoreType.DMA,
        ],
        compiler_params=pltpu.CompilerParams(
            use_tc_tiling_on_sc=True, needs_layout_passes=False
        ),
    )


_BB = 512


def _tc_body(dense_ref, spt_ref,
             dW1, db1, dW2, db2, dW3, db3,
             oW1, ob1, oW2, ob2, oW3, ob3,
             out_ref):
    relu = lambda v: jnp.maximum(v, 0.0)
    h = relu(jnp.dot(dense_ref[...], dW1[...], preferred_element_type=jnp.float32) + db1[...])
    h = relu(jnp.dot(h, dW2[...], preferred_element_type=jnp.float32) + db2[...])
    de = relu(jnp.dot(h, dW3[...], preferred_element_type=jnp.float32) + db3[...])

    sp = jnp.transpose(spt_ref[...])
    c3 = jnp.concatenate([de, sp], axis=1).reshape(_BB, _F + 1, _D)
    g = lax.dot_general(
        c3, c3,
        dimension_numbers=(((2,), (2,)), ((0,), (0,))),
        preferred_element_type=jnp.float32,
    )
    gf = g.reshape(_BB, (_F + 1) * (_F + 1))
    h = relu(
        jnp.dot(de, oW1[: _D], preferred_element_type=jnp.float32)
        + jnp.dot(gf, oW1[_D:], preferred_element_type=jnp.float32)
        + ob1[...]
    )
    h = relu(jnp.dot(h, oW2[...], preferred_element_type=jnp.float32) + ob2[...])
    out_ref[...] = jnp.dot(h, oW3[...], preferred_element_type=jnp.float32) + ob3[...]


def _tc_forward(dense, spt, dW1, db1, dW2, db2, dW3, db3, oW1, ob1, oW2, ob2, oW3, ob3,
                interpret=False):
    full = lambda shape: pl.BlockSpec(shape, lambda i: (0,) * len(shape))
    grid = _B // _BB
    return pl.pallas_call(
        _tc_body,
        grid=(grid,),
        in_specs=[
            pl.BlockSpec((_BB, 13), lambda i: (i, 0)),
            pl.BlockSpec((_F * _D, _BB), lambda i: (0, i)),
            full(dW1.shape), full(db1.shape), full(dW2.shape), full(db2.shape),
            full(dW3.shape), full(db3.shape),
            full(oW1.shape), full(ob1.shape), full(oW2.shape), full(ob2.shape),
            full(oW3.shape), full(ob3.shape),
        ],
        out_specs=pl.BlockSpec((_BB, 1), lambda i: (i, 0)),
        out_shape=jax.ShapeDtypeStruct((_B, 1), jnp.float32),
        interpret=interpret,
    )(dense, spt, dW1, db1, dW2, db2, dW3, db3, oW1, ob1, oW2, ob2, oW3, ob3)


def _permute_over_w1(oW1):
    n_idx, m_idx = jnp.triu_indices(_F + 1, k=1)
    w_int = 0.5 * oW1[_D:]
    wp = jnp.zeros(((_F + 1) * (_F + 1), oW1.shape[1]), oW1.dtype)
    wp = wp.at[n_idx * (_F + 1) + m_idx].set(w_int)
    wp = wp.at[m_idx * (_F + 1) + n_idx].set(w_int)
    return jnp.concatenate([oW1[:_D], wp], axis=0)


def kernel(dense_features, sparse_indices, tables,
           dW1, db1, dW2, db2, dW3, db3,
           oW1, ob1, oW2, ob2, oW3, ob3):
    tab_t = jnp.transpose(tables, (0, 2, 1))
    idx_t = jnp.transpose(sparse_indices)
    out_t = _sc_gather()(tab_t, idx_t)
    spt = out_t.reshape(_F * _D, _B)
    return _tc_forward(
        dense_features, spt,
        dW1, db1.reshape(1, -1), dW2, db2.reshape(1, -1), dW3, db3.reshape(1, -1),
        _permute_over_w1(oW1), ob1.reshape(1, -1),
        oW2, ob2.reshape(1, -1), oW3, ob3.reshape(1, -1),
    )
